# baseline (speedup 1.0000x reference)
import functools

import jax
import jax.numpy as jnp
from jax import lax
from jax.experimental import pallas as pl
from jax.experimental.pallas import tpu as pltpu
from jax.experimental.pallas import tpu_sc as plsc

NUM_CLASSES = 100000
HIDDEN = 64
BATCH = 16384
VOCAB = NUM_CLASSES + 1
VMAIN = (VOCAB // 128) * 128
VTAIL = VOCAB - VMAIN
ROW_BUF = VMAIN + 128

_info = plsc.get_sparse_core_info()
NC, NS, L = _info.num_cores, _info.num_subcores, _info.num_lanes
NW = NC * NS
ROWS_PER_W = HIDDEN // NW
NCHUNK = 4
CHUNK = VMAIN // NCHUNK
_TILES = VMAIN // 128
_CHUNK_TILES = [_TILES // NCHUNK + (1 if q < _TILES % NCHUNK else 0) for q in range(NCHUNK)]
_CHUNK_SIZES = [t * 128 for t in _CHUNK_TILES]
_CHUNK_OFFS = [sum(_CHUNK_SIZES[:q]) for q in range(NCHUNK)]
BLK = 4096
NBLK = BATCH // BLK


def _make_kernel():
  mesh = plsc.VectorSubcoreMesh(core_axis_name="c", subcore_axis_name="s")

  @functools.partial(
      pl.kernel,
      mesh=mesh,
      out_type=jax.ShapeDtypeStruct((HIDDEN, BATCH), jnp.float32),
      compiler_params=pltpu.CompilerParams(needs_layout_passes=False),
      scratch_types=[
          pltpu.VMEM((ROW_BUF,), jnp.float32),
          pltpu.VMEM((BATCH,), jnp.int32),
          pltpu.VMEM((2, BLK), jnp.float32),
          pltpu.SemaphoreType.DMA,
          pltpu.SemaphoreType.DMA,
          pltpu.SemaphoreType.DMA,
      ],
  )
  def col_gather(lab_hbm, tab_t_hbm, tail_t_hbm, out_t_hbm, row_v, lab_v,
                 out_v, sem_row, sem_lab, sem_out):
    wid = lax.axis_index("s") * NC + lax.axis_index("c")
    h0 = wid * ROWS_PER_W

    def start_row(h):
      cps = [
          pltpu.async_copy(
              tab_t_hbm.at[h, pl.ds(off, sz)],
              row_v.at[pl.ds(off, sz)],
              sem_row,
          )
          for off, sz in zip(_CHUNK_OFFS, _CHUNK_SIZES)
      ]
      cps.append(
          pltpu.async_copy(
              tail_t_hbm.at[h], row_v.at[pl.ds(VMAIN, 128)], sem_row
          )
      )
      return cps

    lab_cp = pltpu.async_copy(lab_hbm, lab_v, sem_lab)
    row_cps = start_row(h0)
    lab_cp.wait()
    for cp in row_cps:
      cp.wait()
    out_cps = [None, None]
    for r in range(ROWS_PER_W):
      h = h0 + r
      for b in range(NBLK):
        buf = b % 2
        if out_cps[buf] is not None:
          out_cps[buf].wait()

        @plsc.parallel_loop(0, BLK // L, unroll=8)
        def body(i):
          idx = lab_v[pl.ds(b * BLK + i * L, L)]
          out_v[buf, pl.ds(i * L, L)] = plsc.load_gather(row_v, [idx])

        out_cps[buf] = pltpu.async_copy(
            out_v.at[buf], out_t_hbm.at[h, pl.ds(b * BLK, BLK)], sem_out
        )
      if r + 1 < ROWS_PER_W:
        for cp in start_row(h0 + r + 1):
          cp.wait()
    for cp in out_cps:
      cp.wait()

  return col_gather


_gather = _make_kernel()


@jax.jit
def kernel(labels, embedding_table):
  tab_t = embedding_table.T
  tail_t = jnp.pad(lax.slice(tab_t, (0, VMAIN), (HIDDEN, VOCAB)),
                   ((0, 0), (0, 128 - VTAIL)))
  out_t = _gather(jnp.asarray(labels, jnp.int32), tab_t, tail_t)
  return out_t.T
